# baseline (speedup 1.0000x reference)
import functools

import jax
import jax.numpy as jnp
from jax import lax
from jax.experimental import pallas as pl
from jax.experimental.pallas import tpu as pltpu
from jax.experimental.pallas import tpu_sc as plsc

_N = 10000
_NP = 10240
_E = 320000
_NC = 2
_NS = 16
_NW = _NC * _NS
_EPW = _E // _NW
_B = 80
_H = _B // 2
_NB = _EPW // _B
_RPT = _NP // _NS
_L = 16

_TAKE_DNUMS = lax.GatherDimensionNumbers(
    offset_dims=(), collapsed_slice_dims=(0,), start_index_map=(0,))


def _splat(vec, e):
    return lax.gather(
        vec, jnp.full((_L, 1), e, jnp.int32), _TAKE_DNUMS, (1,),
        mode=lax.GatherScatterMode.PROMISE_IN_BOUNDS)


def _make_mp(D):
    mesh = plsc.VectorSubcoreMesh(
        core_axis_name="c", subcore_axis_name="s",
        num_cores=_NC, num_subcores=_NS)

    @functools.partial(
        pl.kernel,
        out_type=jax.ShapeDtypeStruct((_NC, _NP, D), jnp.float32),
        mesh=mesh,
        scratch_types=[
            pltpu.VMEM((_EPW,), jnp.int32),
            pltpu.VMEM((3, 2, _H), jnp.int32),
            pltpu.VMEM((3, _B), jnp.float32),
            pltpu.VMEM((_B, D), jnp.float32),
            pltpu.VMEM((_B, D), jnp.float32),
            pltpu.VMEM((_B, D), jnp.float32),
            pltpu.VMEM_SHARED((_NP, D), jnp.float32),
            pltpu.SemaphoreType.DMA,
            pltpu.SemaphoreType.DMA,
            pltpu.SemaphoreType.DMA,
            pltpu.SemaphoreType.DMA,
            pltpu.SemaphoreType.DMA,
            pltpu.SemaphoreType.DMA,
        ],
        compiler_params=pltpu.CompilerParams(use_tc_tiling_on_sc=False),
    )
    def mp(h_hbm, src_hbm, dst3_hbm, w_hbm, out_hbm,
           src_v, dst_r, w_r, rows0, rows1, rows2, acc_sh,
           g0, g1, g2, s0, s1, s2):
        c = lax.axis_index("c")
        s = lax.axis_index("s")
        wid = c * _NS + s
        ebase = wid * _EPW
        bufs = ((rows0, g0, s0), (rows1, g1, s1), (rows2, g2, s2))

        def gather(i, u, sem):
            rows = bufs[u][0]
            pltpu.async_copy(h_hbm.at[src_v.at[pl.ds(i * _B, _B)]], rows, sem)
            pltpu.async_copy(w_hbm.at[pl.ds(ebase + i * _B, _B)],
                             w_r.at[u], sem)
            pltpu.async_copy(dst3_hbm.at[wid, i], dst_r.at[u], sem)

        def gwait(u, sem):
            pltpu.make_async_copy(h_hbm.at[pl.ds(0, _B)], bufs[u][0],
                                  sem).wait()
            pltpu.make_async_copy(w_hbm.at[pl.ds(0, _B)], w_r.at[u],
                                  sem).wait()
            pltpu.make_async_copy(dst3_hbm.at[0, 0], dst_r.at[u],
                                  sem).wait()

        def swait(u, sem):
            pltpu.make_async_copy(h_hbm.at[pl.ds(0, _H)],
                                  bufs[u][0].at[pl.ds(0, _H)], sem).wait()

        def scale_edges(u, rows, k0, k1):
            for g in range(k0 // _L, (k1 + _L - 1) // _L):
                lo = max(k0 - g * _L, 0)
                hi = min(k1 - g * _L, _L)
                wv = w_r[u, pl.ds(g * _L, _L)]
                for e in range(lo, hi):
                    k = g * _L + e
                    wk = _splat(wv, e)
                    for j in range(D // _L):
                        sl = (k, pl.ds(j * _L, _L))
                        rows[sl] = rows[sl] * wk

        pltpu.sync_copy(src_hbm.at[pl.ds(ebase, _EPW)], src_v)
        pltpu.sync_copy(dst3_hbm.at[wid, 0], dst_r.at[2])
        gather(0, 0, g0)
        gather(1, 1, g1)

        zero = jnp.zeros((_L,), jnp.float32)
        for k in range(_B):
            for j in range(D // _L):
                rows2[k, pl.ds(j * _L, _L)] = zero
        for m in range(_RPT // _B):
            pltpu.async_copy(rows2, acc_sh.at[pl.ds(s * _RPT + m * _B, _B)],
                             s0)
        for m in range(_RPT // _B):
            pltpu.make_async_copy(h_hbm.at[pl.ds(0, _B)], rows2, s0).wait()
        plsc.subcore_barrier()
        pltpu.async_copy(rows2.at[pl.ds(0, _H)], acc_sh.at[dst_r.at[2, 0]],
                         s2, add=True)


        def body(t, carry):
            for u in range(3):
                bi = 3 * t + u
                rows, gs, ss = bufs[u]
                u2 = (u + 2) % 3
                ngs, nss = bufs[u2][1], bufs[u2][2]

                @pl.when(bi < _NB)
                def _():
                    gwait(u, gs)
                    scale_edges(u, rows, 0, _H)
                    swait(u2, nss)
                    pltpu.async_copy(rows.at[pl.ds(0, _H)],
                                     acc_sh.at[dst_r.at[u, 0]], ss, add=True)
                    scale_edges(u, rows, _H, _B)
                    swait(u, ss)
                    pltpu.async_copy(rows.at[pl.ds(_H, _H)],
                                     acc_sh.at[dst_r.at[u, 1]], ss, add=True)

                    @pl.when(bi + 2 < _NB)
                    def _():
                        gather(bi + 2, u2, ngs)

            return carry

        lax.fori_loop(0, (_NB + 2) // 3, body, 0)
        swait((_NB - 1) % 3, bufs[(_NB - 1) % 3][2])
        plsc.subcore_barrier()

        pltpu.sync_copy(acc_sh.at[pl.ds(s * _RPT, _RPT)],
                        out_hbm.at[c, pl.ds(s * _RPT, _RPT)])

    return mp


_mp128 = _make_mp(128)
_mp64 = _make_mp(64)

_BLK = 1000
_BLKP = 640


def _mm1_body(x_ref, w_ref, o_ref):
    o_ref[...] = jnp.dot(x_ref[...], w_ref[...],
                         preferred_element_type=jnp.float32)


def _mm2_body(p_ref, b_ref, w_ref, o_ref):
    z = jnp.maximum(p_ref[0] + p_ref[1] + b_ref[...], 0.0)
    o_ref[...] = jnp.dot(z, w_ref[...], preferred_element_type=jnp.float32)


def _lsm_body(q_ref, b_ref, o_ref):
    t = q_ref[0] + q_ref[1] + b_ref[...]
    m = jnp.max(t, axis=1, keepdims=True)
    e = jnp.exp(t - m)
    lse = jnp.log(jnp.sum(e, axis=1, keepdims=True))
    o_ref[...] = t - m - lse


def kernel(x, edge_index, edge_weight, W1, b1, W2, b2):
    src = edge_index[0]
    dst3 = edge_index[1].reshape(_NW, _NB, 2, _H)
    f_in = x.shape[1]
    f_hid = W1.shape[1]
    f_out = W2.shape[1]
    nblk = _N // _BLK

    h = pl.pallas_call(
        _mm1_body,
        grid=(nblk,),
        in_specs=[
            pl.BlockSpec((_BLK, f_in), lambda i: (i, 0)),
            pl.BlockSpec((f_in, f_hid), lambda i: (0, 0)),
        ],
        out_specs=pl.BlockSpec((_BLK, f_hid), lambda i: (i, 0)),
        out_shape=jax.ShapeDtypeStruct((_N, f_hid), jnp.float32),
    )(x, W1)

    p = _mp128(h, src, dst3, edge_weight)

    nblkp = _NP // _BLKP
    h2 = pl.pallas_call(
        _mm2_body,
        grid=(nblkp,),
        in_specs=[
            pl.BlockSpec((2, _BLKP, f_hid), lambda i: (0, i, 0)),
            pl.BlockSpec((1, f_hid), lambda i: (0, 0)),
            pl.BlockSpec((f_hid, f_out), lambda i: (0, 0)),
        ],
        out_specs=pl.BlockSpec((_BLKP, f_out), lambda i: (i, 0)),
        out_shape=jax.ShapeDtypeStruct((_NP, f_out), jnp.float32),
    )(p, b1.reshape(1, f_hid), W2)

    q = _mp64(h2, src, dst3, edge_weight)

    out = pl.pallas_call(
        _lsm_body,
        grid=(nblkp,),
        in_specs=[
            pl.BlockSpec((2, _BLKP, f_out), lambda i: (0, i, 0)),
            pl.BlockSpec((1, f_out), lambda i: (0, 0)),
        ],
        out_specs=pl.BlockSpec((_BLKP, f_out), lambda i: (i, 0)),
        out_shape=jax.ShapeDtypeStruct((_NP, f_out), jnp.float32),
    )(q, b2.reshape(1, f_out))

    return out[:_N]


# speedup vs baseline: 1.0239x; 1.0239x over previous
import functools

import jax
import jax.numpy as jnp
from jax import lax
from jax.experimental import pallas as pl
from jax.experimental.pallas import tpu as pltpu
from jax.experimental.pallas import tpu_sc as plsc

_N = 10000
_NP = 10240
_E = 320000
_NC = 2
_NS = 16
_NW = _NC * _NS
_EPW = _E // _NW
_B = 80
_H = _B // 2
_NB = _EPW // _B
_RPT = _NP // _NS
_L = 16

_TAKE_DNUMS = lax.GatherDimensionNumbers(
    offset_dims=(), collapsed_slice_dims=(0,), start_index_map=(0,))


def _splat(vec, e):
    return lax.gather(
        vec, jnp.full((_L, 1), e, jnp.int32), _TAKE_DNUMS, (1,),
        mode=lax.GatherScatterMode.PROMISE_IN_BOUNDS)


def _make_mp(D):
    mesh = plsc.VectorSubcoreMesh(
        core_axis_name="c", subcore_axis_name="s",
        num_cores=_NC, num_subcores=_NS)

    @functools.partial(
        pl.kernel,
        out_type=jax.ShapeDtypeStruct((_NC, _NP, D), jnp.float32),
        mesh=mesh,
        scratch_types=[
            pltpu.VMEM((_EPW,), jnp.int32),
            pltpu.VMEM((3, _B), jnp.int32),
            pltpu.VMEM((3, _B), jnp.float32),
            pltpu.VMEM((_B, D), jnp.float32),
            pltpu.VMEM((_B, D), jnp.float32),
            pltpu.VMEM((_B, D), jnp.float32),
            pltpu.VMEM_SHARED((_NP, D), jnp.float32),
            pltpu.SemaphoreType.DMA,
            pltpu.SemaphoreType.DMA,
            pltpu.SemaphoreType.DMA,
            pltpu.SemaphoreType.DMA,
            pltpu.SemaphoreType.DMA,
            pltpu.SemaphoreType.DMA,
        ],
        compiler_params=pltpu.CompilerParams(use_tc_tiling_on_sc=False),
    )
    def mp(h_hbm, src_hbm, dst3_hbm, w_hbm, out_hbm,
           src_v, dst_r, w_r, rows0, rows1, rows2, acc_sh,
           g0, g1, g2, s0, s1, s2):
        c = lax.axis_index("c")
        s = lax.axis_index("s")
        wid = c * _NS + s
        ebase = wid * _EPW
        bufs = ((rows0, g0, s0), (rows1, g1, s1), (rows2, g2, s2))

        def gather(i, u, sem):
            rows = bufs[u][0]
            pltpu.async_copy(h_hbm.at[src_v.at[pl.ds(i * _B, _B)]], rows, sem)
            pltpu.async_copy(w_hbm.at[pl.ds(ebase + i * _B, _B)],
                             w_r.at[u], sem)
            pltpu.async_copy(dst3_hbm.at[wid, i], dst_r.at[u], sem)

        def gwait(u, sem):
            pltpu.make_async_copy(h_hbm.at[pl.ds(0, _B)], bufs[u][0],
                                  sem).wait()
            pltpu.make_async_copy(w_hbm.at[pl.ds(0, _B)], w_r.at[u],
                                  sem).wait()
            pltpu.make_async_copy(dst3_hbm.at[0, 0], dst_r.at[u],
                                  sem).wait()

        def swait(u, sem):
            pltpu.make_async_copy(h_hbm.at[pl.ds(0, _B)], bufs[u][0],
                                  sem).wait()

        def scale_edges(u, rows, k0, k1):
            for g in range(k0 // _L, (k1 + _L - 1) // _L):
                lo = max(k0 - g * _L, 0)
                hi = min(k1 - g * _L, _L)
                wv = w_r[u, pl.ds(g * _L, _L)]
                for e in range(lo, hi):
                    k = g * _L + e
                    wk = _splat(wv, e)
                    for j in range(D // _L):
                        sl = (k, pl.ds(j * _L, _L))
                        rows[sl] = rows[sl] * wk

        pltpu.sync_copy(src_hbm.at[pl.ds(ebase, _EPW)], src_v)
        pltpu.sync_copy(dst3_hbm.at[wid, 0], dst_r.at[2])
        gather(0, 0, g0)
        gather(1, 1, g1)

        zero = jnp.zeros((_L,), jnp.float32)
        for k in range(_B):
            for j in range(D // _L):
                rows2[k, pl.ds(j * _L, _L)] = zero
        for m in range(_RPT // _B):
            pltpu.async_copy(rows2, acc_sh.at[pl.ds(s * _RPT + m * _B, _B)],
                             s0)
        for m in range(_RPT // _B):
            pltpu.make_async_copy(h_hbm.at[pl.ds(0, _B)], rows2, s0).wait()
        plsc.subcore_barrier()
        pltpu.async_copy(rows2, acc_sh.at[dst_r.at[2]], s2, add=True)


        def body(t, carry):
            for u in range(3):
                bi = 3 * t + u
                rows, gs, ss = bufs[u]
                u2 = (u + 2) % 3
                ngs, nss = bufs[u2][1], bufs[u2][2]

                @pl.when(bi < _NB)
                def _():
                    gwait(u, gs)
                    scale_edges(u, rows, 0, _B)
                    swait(u2, nss)
                    pltpu.async_copy(rows, acc_sh.at[dst_r.at[u]],
                                     ss, add=True)

                    @pl.when(bi + 2 < _NB)
                    def _():
                        gather(bi + 2, u2, ngs)

            return carry

        lax.fori_loop(0, (_NB + 2) // 3, body, 0)
        swait((_NB - 1) % 3, bufs[(_NB - 1) % 3][2])
        plsc.subcore_barrier()

        pltpu.sync_copy(acc_sh.at[pl.ds(s * _RPT, _RPT)],
                        out_hbm.at[c, pl.ds(s * _RPT, _RPT)])

    return mp


_mp128 = _make_mp(128)
_mp64 = _make_mp(64)

_BLK = 1000
_BLKP = 640


def _mm1_body(x_ref, w_ref, o_ref):
    o_ref[...] = jnp.dot(x_ref[...], w_ref[...],
                         preferred_element_type=jnp.float32)


def _mm2_body(p_ref, b_ref, w_ref, o_ref):
    z = jnp.maximum(p_ref[0] + p_ref[1] + b_ref[...], 0.0)
    o_ref[...] = jnp.dot(z, w_ref[...], preferred_element_type=jnp.float32)


def _lsm_body(q_ref, b_ref, o_ref):
    t = q_ref[0] + q_ref[1] + b_ref[...]
    m = jnp.max(t, axis=1, keepdims=True)
    e = jnp.exp(t - m)
    lse = jnp.log(jnp.sum(e, axis=1, keepdims=True))
    o_ref[...] = t - m - lse


def kernel(x, edge_index, edge_weight, W1, b1, W2, b2):
    src = edge_index[0]
    dst3 = edge_index[1].reshape(_NW, _NB, _B)
    f_in = x.shape[1]
    f_hid = W1.shape[1]
    f_out = W2.shape[1]
    nblk = _N // _BLK

    h = pl.pallas_call(
        _mm1_body,
        grid=(nblk,),
        in_specs=[
            pl.BlockSpec((_BLK, f_in), lambda i: (i, 0)),
            pl.BlockSpec((f_in, f_hid), lambda i: (0, 0)),
        ],
        out_specs=pl.BlockSpec((_BLK, f_hid), lambda i: (i, 0)),
        out_shape=jax.ShapeDtypeStruct((_N, f_hid), jnp.float32),
    )(x, W1)

    p = _mp128(h, src, dst3, edge_weight)

    nblkp = _NP // _BLKP
    h2 = pl.pallas_call(
        _mm2_body,
        grid=(nblkp,),
        in_specs=[
            pl.BlockSpec((2, _BLKP, f_hid), lambda i: (0, i, 0)),
            pl.BlockSpec((1, f_hid), lambda i: (0, 0)),
            pl.BlockSpec((f_hid, f_out), lambda i: (0, 0)),
        ],
        out_specs=pl.BlockSpec((_BLKP, f_out), lambda i: (i, 0)),
        out_shape=jax.ShapeDtypeStruct((_NP, f_out), jnp.float32),
    )(p, b1.reshape(1, f_hid), W2)

    q = _mp64(h2, src, dst3, edge_weight)

    out = pl.pallas_call(
        _lsm_body,
        grid=(nblkp,),
        in_specs=[
            pl.BlockSpec((2, _BLKP, f_out), lambda i: (0, i, 0)),
            pl.BlockSpec((1, f_out), lambda i: (0, 0)),
        ],
        out_specs=pl.BlockSpec((_BLKP, f_out), lambda i: (i, 0)),
        out_shape=jax.ShapeDtypeStruct((_NP, f_out), jnp.float32),
    )(q, b2.reshape(1, f_out))

    return out[:_N]


# speedup vs baseline: 1.0544x; 1.0298x over previous
import functools

import jax
import jax.numpy as jnp
from jax import lax
from jax.experimental import pallas as pl
from jax.experimental.pallas import tpu as pltpu
from jax.experimental.pallas import tpu_sc as plsc

_N = 10000
_NP = 10240
_E = 320000
_NC = 2
_NS = 16
_NW = _NC * _NS
_EPW = _E // _NW
_B = 80
_H = _B // 2
_NB = _EPW // _B
_RPT = _NP // _NS
_L = 16

_TAKE_DNUMS = lax.GatherDimensionNumbers(
    offset_dims=(), collapsed_slice_dims=(0,), start_index_map=(0,))


def _splat(vec, e):
    return lax.gather(
        vec, jnp.full((_L, 1), e, jnp.int32), _TAKE_DNUMS, (1,),
        mode=lax.GatherScatterMode.PROMISE_IN_BOUNDS)


def _make_mp(D):
    mesh = plsc.VectorSubcoreMesh(
        core_axis_name="c", subcore_axis_name="s",
        num_cores=_NC, num_subcores=_NS)

    @functools.partial(
        pl.kernel,
        out_type=jax.ShapeDtypeStruct((_NC, _NP, D), jnp.float32),
        mesh=mesh,
        scratch_types=[
            pltpu.VMEM((_EPW,), jnp.int32),
            pltpu.VMEM((3, _B), jnp.int32),
            pltpu.VMEM((3, _B), jnp.float32),
            pltpu.VMEM((_B, D), jnp.float32),
            pltpu.VMEM((_B, D), jnp.float32),
            pltpu.VMEM((_B, D), jnp.float32),
            pltpu.VMEM_SHARED((_NP, D), jnp.float32),
            pltpu.SemaphoreType.DMA,
            pltpu.SemaphoreType.DMA,
            pltpu.SemaphoreType.DMA,
            pltpu.SemaphoreType.DMA,
            pltpu.SemaphoreType.DMA,
            pltpu.SemaphoreType.DMA,
        ],
        compiler_params=pltpu.CompilerParams(use_tc_tiling_on_sc=False),
    )
    def mp(h_hbm, src_hbm, dst3_hbm, w_hbm, out_hbm,
           src_v, dst_r, w_r, rows0, rows1, rows2, acc_sh,
           g0, g1, g2, s0, s1, s2):
        c = lax.axis_index("c")
        s = lax.axis_index("s")
        wid = c * _NS + s
        ebase = wid * _EPW
        bufs = ((rows0, g0, s0), (rows1, g1, s1), (rows2, g2, s2))

        def gather(i, u, sem):
            rows = bufs[u][0]
            pltpu.async_copy(h_hbm.at[src_v.at[pl.ds(i * _B, _B)]], rows, sem)
            pltpu.async_copy(w_hbm.at[pl.ds(ebase + i * _B, _B)],
                             w_r.at[u], sem)
            pltpu.async_copy(dst3_hbm.at[wid, i], dst_r.at[u], sem)

        def gwait(u, sem):
            pltpu.make_async_copy(h_hbm.at[pl.ds(0, _B)], bufs[u][0],
                                  sem).wait()
            pltpu.make_async_copy(w_hbm.at[pl.ds(0, _B)], w_r.at[u],
                                  sem).wait()
            pltpu.make_async_copy(dst3_hbm.at[0, 0], dst_r.at[u],
                                  sem).wait()

        def swait(u, sem):
            pltpu.make_async_copy(h_hbm.at[pl.ds(0, _B)], bufs[u][0],
                                  sem).wait()

        def scale_edges(u, rows, k0, k1):
            for g in range(k0 // _L, (k1 + _L - 1) // _L):
                lo = max(k0 - g * _L, 0)
                hi = min(k1 - g * _L, _L)
                wv = w_r[u, pl.ds(g * _L, _L)]
                for e in range(lo, hi):
                    k = g * _L + e
                    wk = _splat(wv, e)
                    for j in range(D // _L):
                        sl = (k, pl.ds(j * _L, _L))
                        rows[sl] = rows[sl] * wk

        pltpu.sync_copy(src_hbm.at[pl.ds(ebase, _EPW)], src_v)
        pltpu.sync_copy(dst3_hbm.at[wid, 0], dst_r.at[2])
        gather(0, 0, g0)
        gather(1, 1, g1)

        zero = jnp.zeros((_L,), jnp.float32)
        for k in range(_B):
            for j in range(D // _L):
                rows2[k, pl.ds(j * _L, _L)] = zero
        for m in range(_RPT // _B):
            pltpu.async_copy(rows2, acc_sh.at[pl.ds(s * _RPT + m * _B, _B)],
                             s0)
        for m in range(_RPT // _B):
            pltpu.make_async_copy(h_hbm.at[pl.ds(0, _B)], rows2, s0).wait()
        plsc.subcore_barrier()
        pltpu.async_copy(rows2, acc_sh.at[dst_r.at[2]], s2, add=True)


        def body(t, carry):
            for u in range(3):
                bi = 3 * t + u
                rows, gs, ss = bufs[u]
                u2 = (u + 2) % 3
                ngs, nss = bufs[u2][1], bufs[u2][2]

                @pl.when(bi < _NB)
                def _():
                    gwait(u, gs)
                    scale_edges(u, rows, 0, _B)
                    swait(u2, nss)
                    pltpu.async_copy(rows, acc_sh.at[dst_r.at[u]],
                                     ss, add=True)

                    @pl.when(bi + 2 < _NB)
                    def _():
                        gather(bi + 2, u2, ngs)

            return carry

        lax.fori_loop(0, (_NB + 2) // 3, body, 0)
        swait((_NB - 1) % 3, bufs[(_NB - 1) % 3][2])
        plsc.subcore_barrier()

        pltpu.sync_copy(acc_sh.at[pl.ds(s * _RPT, _RPT)],
                        out_hbm.at[c, pl.ds(s * _RPT, _RPT)])

    return mp


_mp128 = _make_mp(128)
_mp64 = _make_mp(64)

_BLK = 2000
_BLKP = 1280


def _mm1_body(x_ref, w_ref, o_ref):
    o_ref[...] = jnp.dot(x_ref[...], w_ref[...],
                         preferred_element_type=jnp.float32)


def _mm2_body(p_ref, b_ref, w_ref, o_ref):
    z = jnp.maximum(p_ref[0] + p_ref[1] + b_ref[...], 0.0)
    o_ref[...] = jnp.dot(z, w_ref[...], preferred_element_type=jnp.float32)


def _lsm_body(q_ref, b_ref, o_ref):
    t = q_ref[0] + q_ref[1] + b_ref[...]
    m = jnp.max(t, axis=1, keepdims=True)
    e = jnp.exp(t - m)
    lse = jnp.log(jnp.sum(e, axis=1, keepdims=True))
    o_ref[...] = t - m - lse


def kernel(x, edge_index, edge_weight, W1, b1, W2, b2):
    src = edge_index[0]
    dst3 = edge_index[1].reshape(_NW, _NB, _B)
    f_in = x.shape[1]
    f_hid = W1.shape[1]
    f_out = W2.shape[1]
    nblk = _N // _BLK

    h = pl.pallas_call(
        _mm1_body,
        grid=(nblk,),
        in_specs=[
            pl.BlockSpec((_BLK, f_in), lambda i: (i, 0)),
            pl.BlockSpec((f_in, f_hid), lambda i: (0, 0)),
        ],
        out_specs=pl.BlockSpec((_BLK, f_hid), lambda i: (i, 0)),
        out_shape=jax.ShapeDtypeStruct((_N, f_hid), jnp.float32),
    )(x, W1)

    p = _mp128(h, src, dst3, edge_weight)

    nblkp = _NP // _BLKP
    h2 = pl.pallas_call(
        _mm2_body,
        grid=(nblkp,),
        in_specs=[
            pl.BlockSpec((2, _BLKP, f_hid), lambda i: (0, i, 0)),
            pl.BlockSpec((1, f_hid), lambda i: (0, 0)),
            pl.BlockSpec((f_hid, f_out), lambda i: (0, 0)),
        ],
        out_specs=pl.BlockSpec((_BLKP, f_out), lambda i: (i, 0)),
        out_shape=jax.ShapeDtypeStruct((_NP, f_out), jnp.float32),
    )(p, b1.reshape(1, f_hid), W2)

    q = _mp64(h2, src, dst3, edge_weight)

    out = pl.pallas_call(
        _lsm_body,
        grid=(nblkp,),
        in_specs=[
            pl.BlockSpec((2, _BLKP, f_out), lambda i: (0, i, 0)),
            pl.BlockSpec((1, f_out), lambda i: (0, 0)),
        ],
        out_specs=pl.BlockSpec((_BLKP, f_out), lambda i: (i, 0)),
        out_shape=jax.ShapeDtypeStruct((_N, f_out), jnp.float32),
    )(q, b2.reshape(1, f_out))

    return out


# speedup vs baseline: 1.0574x; 1.0029x over previous
import functools

import jax
import jax.numpy as jnp
from jax import lax
from jax.experimental import pallas as pl
from jax.experimental.pallas import tpu as pltpu
from jax.experimental.pallas import tpu_sc as plsc

_N = 10000
_NP = 10240
_E = 320000
_NC = 2
_NS = 16
_NW = _NC * _NS
_EPW = _E // _NW
_B = 80
_NB = _EPW // _B
_RPT = _NP // _NS
_L = 16

_TAKE_DNUMS = lax.GatherDimensionNumbers(
    offset_dims=(), collapsed_slice_dims=(0,), start_index_map=(0,))


def _splat(vec, e):
    return lax.gather(
        vec, jnp.full((_L, 1), e, jnp.int32), _TAKE_DNUMS, (1,),
        mode=lax.GatherScatterMode.PROMISE_IN_BOUNDS)


def _make_mp(D):
    mesh = plsc.VectorSubcoreMesh(
        core_axis_name="c", subcore_axis_name="s",
        num_cores=_NC, num_subcores=_NS)

    @functools.partial(
        pl.kernel,
        out_type=jax.ShapeDtypeStruct((_NC, _NP, D), jnp.float32),
        mesh=mesh,
        scratch_types=[
            pltpu.VMEM((_EPW,), jnp.int32),
            pltpu.VMEM((3, _B), jnp.int32),
            pltpu.VMEM((3, _B), jnp.float32),
            pltpu.VMEM((_B, D), jnp.float32),
            pltpu.VMEM((_B, D), jnp.float32),
            pltpu.VMEM((_B, D), jnp.float32),
            pltpu.VMEM_SHARED((_NP, D), jnp.float32),
            pltpu.SemaphoreType.DMA,
            pltpu.SemaphoreType.DMA,
            pltpu.SemaphoreType.DMA,
            pltpu.SemaphoreType.DMA,
            pltpu.SemaphoreType.DMA,
            pltpu.SemaphoreType.DMA,
        ],
        compiler_params=pltpu.CompilerParams(use_tc_tiling_on_sc=False),
    )
    def mp(h_hbm, src_hbm, dst3_hbm, w_hbm, out_hbm,
           src_v, dst_r, w_r, rows0, rows1, rows2, acc_sh,
           g0, g1, g2, s0, s1, s2):
        c = lax.axis_index("c")
        s = lax.axis_index("s")
        wid = c * _NS + s
        ebase = wid * _EPW
        bufs = ((rows0, g0, s0), (rows1, g1, s1), (rows2, g2, s2))

        def gather(i, u, sem):
            rows = bufs[u][0]
            pltpu.async_copy(h_hbm.at[src_v.at[pl.ds(i * _B, _B)]], rows, sem)
            pltpu.async_copy(w_hbm.at[pl.ds(ebase + i * _B, _B)],
                             w_r.at[u], sem)
            pltpu.async_copy(dst3_hbm.at[wid, i], dst_r.at[u], sem)

        def gwait(u, sem):
            pltpu.make_async_copy(h_hbm.at[pl.ds(0, _B)], bufs[u][0],
                                  sem).wait()
            pltpu.make_async_copy(w_hbm.at[pl.ds(0, _B)], w_r.at[u],
                                  sem).wait()
            pltpu.make_async_copy(dst3_hbm.at[0, 0], dst_r.at[u],
                                  sem).wait()

        def swait(u, sem):
            pltpu.make_async_copy(h_hbm.at[pl.ds(0, _B)], bufs[u][0],
                                  sem).wait()

        def scale_edges(u, rows, k0, k1):
            for g in range(k0 // _L, (k1 + _L - 1) // _L):
                lo = max(k0 - g * _L, 0)
                hi = min(k1 - g * _L, _L)
                wv = w_r[u, pl.ds(g * _L, _L)]
                for e in range(lo, hi):
                    k = g * _L + e
                    wk = _splat(wv, e)
                    for j in range(D // _L):
                        sl = (k, pl.ds(j * _L, _L))
                        rows[sl] = rows[sl] * wk

        pltpu.sync_copy(src_hbm.at[pl.ds(ebase, _EPW)], src_v)
        pltpu.sync_copy(dst3_hbm.at[wid, 0], dst_r.at[2])
        gather(0, 0, g0)
        gather(1, 1, g1)

        zero = jnp.zeros((_L,), jnp.float32)
        for k in range(_B):
            for j in range(D // _L):
                rows2[k, pl.ds(j * _L, _L)] = zero
        for m in range(_RPT // _B):
            pltpu.async_copy(rows2, acc_sh.at[pl.ds(s * _RPT + m * _B, _B)],
                             s0)
        for m in range(_RPT // _B):
            pltpu.make_async_copy(h_hbm.at[pl.ds(0, _B)], rows2, s0).wait()
        plsc.subcore_barrier()
        pltpu.async_copy(rows2, acc_sh.at[dst_r.at[2]], s2, add=True)


        def body(t, carry):
            for u in range(3):
                bi = 3 * t + u
                rows, gs, ss = bufs[u]
                u2 = (u + 2) % 3
                ngs, nss = bufs[u2][1], bufs[u2][2]

                @pl.when(bi < _NB)
                def _():
                    gwait(u, gs)
                    scale_edges(u, rows, 0, _B)
                    swait(u2, nss)
                    pltpu.async_copy(rows, acc_sh.at[dst_r.at[u]],
                                     ss, add=True)

                    @pl.when(bi + 2 < _NB)
                    def _():
                        gather(bi + 2, u2, ngs)

            return carry

        lax.fori_loop(0, (_NB + 2) // 3, body, 0)
        swait((_NB - 1) % 3, bufs[(_NB - 1) % 3][2])
        plsc.subcore_barrier()

        pltpu.sync_copy(acc_sh.at[pl.ds(s * _RPT, _RPT)],
                        out_hbm.at[c, pl.ds(s * _RPT, _RPT)])

    return mp


_mp128 = _make_mp(128)
_mp64 = _make_mp(64)

_BLK = 2000
_BLKP = 1280


def _mm1_body(x_ref, w_ref, o_ref):
    o_ref[...] = jnp.dot(x_ref[...], w_ref[...],
                         preferred_element_type=jnp.float32)


def _mm2_body(p_ref, b_ref, w_ref, o_ref):
    z = jnp.maximum(p_ref[0] + p_ref[1] + b_ref[...], 0.0)
    o_ref[...] = jnp.dot(z, w_ref[...], preferred_element_type=jnp.float32)


def _lsm_body(q_ref, b_ref, o_ref):
    t = q_ref[0] + q_ref[1] + b_ref[...]
    m = jnp.max(t, axis=1, keepdims=True)
    e = jnp.exp(t - m)
    lse = jnp.log(jnp.sum(e, axis=1, keepdims=True))
    o_ref[...] = t - m - lse


def kernel(x, edge_index, edge_weight, W1, b1, W2, b2):
    src = edge_index[0]
    dst3 = edge_index[1].reshape(_NW, _NB, _B)
    f_in = x.shape[1]
    f_hid = W1.shape[1]
    f_out = W2.shape[1]
    nblk = _N // _BLK

    h = pl.pallas_call(
        _mm1_body,
        grid=(nblk,),
        in_specs=[
            pl.BlockSpec((_BLK, f_in), lambda i: (i, 0)),
            pl.BlockSpec((f_in, f_hid), lambda i: (0, 0)),
        ],
        out_specs=pl.BlockSpec((_BLK, f_hid), lambda i: (i, 0)),
        out_shape=jax.ShapeDtypeStruct((_N, f_hid), jnp.float32),
    )(x, W1)

    p = _mp128(h, src, dst3, edge_weight)

    nblkp = _NP // _BLKP
    h2 = pl.pallas_call(
        _mm2_body,
        grid=(nblkp,),
        in_specs=[
            pl.BlockSpec((2, _BLKP, f_hid), lambda i: (0, i, 0)),
            pl.BlockSpec((1, f_hid), lambda i: (0, 0)),
            pl.BlockSpec((f_hid, f_out), lambda i: (0, 0)),
        ],
        out_specs=pl.BlockSpec((_BLKP, f_out), lambda i: (i, 0)),
        out_shape=jax.ShapeDtypeStruct((_NP, f_out), jnp.float32),
    )(p, b1.reshape(1, f_hid), W2)

    q = _mp64(h2, src, dst3, edge_weight)

    out = pl.pallas_call(
        _lsm_body,
        grid=(nblkp,),
        in_specs=[
            pl.BlockSpec((2, _BLKP, f_out), lambda i: (0, i, 0)),
            pl.BlockSpec((1, f_out), lambda i: (0, 0)),
        ],
        out_specs=pl.BlockSpec((_BLKP, f_out), lambda i: (i, 0)),
        out_shape=jax.ShapeDtypeStruct((_N, f_out), jnp.float32),
    )(q, b2.reshape(1, f_out))

    return out
